# Initial kernel scaffold; baseline (speedup 1.0000x reference)
import jax
import jax.numpy as jnp
from jax.experimental import pallas as pl


def kernel(input_ids, input_mask, embedding_table):
    raise NotImplementedError("write your pallas kernel here")


# speedup vs baseline: 1.1314x; 1.1314x over previous
import functools

import jax
import jax.numpy as jnp
from jax import lax
from jax.experimental import pallas as pl
from jax.experimental.pallas import tpu as pltpu
from jax.experimental.pallas import tpu_sc as plsc

_NC = 2
_NS = 16
_L = 16


@functools.lru_cache(maxsize=None)
def _make_kernel(N, V, D, C):
    NW = _NC * _NS
    per_w = N // NW
    nchunk = per_w // C
    mesh = plsc.VectorSubcoreMesh(core_axis_name="c", subcore_axis_name="s")

    @functools.partial(
        pl.kernel,
        mesh=mesh,
        out_type=jax.ShapeDtypeStruct((N, D), jnp.float32),
        scratch_types=[
            pltpu.VMEM((per_w,), jnp.int32),
            pltpu.VMEM((per_w,), jnp.int32),
            pltpu.VMEM((C, D), jnp.float32),
            pltpu.SemaphoreType.DMA,
        ],
    )
    def k(ids_hbm, mask_hbm, table_hbm, out_hbm, idx_v, msk_v, rows_v, sem):
        wid = lax.axis_index("s") * _NC + lax.axis_index("c")
        base = wid * per_w
        pltpu.sync_copy(ids_hbm.at[pl.ds(base, per_w)], idx_v)
        pltpu.sync_copy(mask_hbm.at[pl.ds(base, per_w)], msk_v)

        def mul_body(i, _):
            s = pl.ds(i * _L, _L)
            idx_v[s] = idx_v[s] * msk_v[s]
            return 0

        lax.fori_loop(0, per_w // _L, mul_body, 0, unroll=4)

        def chunk_body(c, _):
            start = c * C
            pltpu.async_copy(
                table_hbm.at[idx_v.at[pl.ds(start, C)]], rows_v, sem
            ).wait()

            for ii in range(C // _L):
                mvec = msk_v[pl.ds(start + ii * _L, _L)]
                for i2 in range(_L):
                    i = ii * _L + i2

                    @pl.when(mvec[i2] == 0)
                    def _zero(i=i):
                        def col_body(j, _):
                            rows_v[i, pl.ds(j * _L, _L)] = jnp.zeros((_L,), jnp.float32)
                            return 0

                        lax.fori_loop(0, D // _L, col_body, 0, unroll=8)
            pltpu.sync_copy(rows_v, out_hbm.at[pl.ds(base + start, C)])
            return 0

        lax.fori_loop(0, nchunk, chunk_body, 0)

    return k


@jax.jit
def kernel(input_ids, input_mask, embedding_table):
    B, S = input_ids.shape
    V, D = embedding_table.shape
    N = B * S
    ids = input_ids.reshape(N)
    msk = input_mask.reshape(N)
    out = _make_kernel(N, V, D, 32)(ids, msk, embedding_table)
    return out.reshape(B, S, D)
